# baseline (speedup 1.0000x reference)
import jax
import jax.numpy as jnp
from jax.experimental import pallas as pl
from jax.experimental.pallas import tpu as pltpu

_N = 8
_D = 32
_H = 32
_T = 4
_B_BLK = 1024
_PK = 128 // _D
_RPC = _N // _PK


def _score_kernel(x_ref, w1b_ref, b1_ref, whb_ref, tr_ref, rot_ref, tor_ref):
    h = jnp.dot(x_ref[...], w1b_ref[...], preferred_element_type=jnp.float32)
    h = jnp.maximum(h + b1_ref[...], 0.0)
    s2 = jnp.sum(h.reshape(_B_BLK, _RPC, 128), axis=1)
    out = jnp.dot(s2, whb_ref[...], preferred_element_type=jnp.float32)
    tr_ref[...] = out[:, 0:3]
    rot_ref[...] = out[:, 3:6]
    tor_ref[...] = out[:, 6:6 + _T]


@jax.jit
def _forward(x, w1_aug, w_heads):
    b = x.shape[0]
    n_blocks = pl.cdiv(b, _B_BLK)
    b_pad = n_blocks * _B_BLK
    if b_pad != b:
        x = jnp.pad(x, ((0, b_pad - b), (0, 0), (0, 0)))

    xp = x.reshape(b_pad * _RPC, _PK * _D)

    w1 = w1_aug[:_D, :]
    b1 = w1_aug[_D:_D + 1, :]
    w1_blk = jnp.kron(jnp.eye(_PK, dtype=w1.dtype), w1)
    b1_t = jnp.tile(b1, (1, _PK))
    wh_stack = jnp.tile(w_heads, (_PK, 1))

    rows = b_pad * _RPC
    flops = 2 * rows * 128 * 128 + 2 * b_pad * 128 * 128
    bytes_accessed = 4 * (rows * 128 + 3 * 128 * 128 + b_pad * (3 + 3 + _T))

    tr, rot, tor = pl.pallas_call(
        _score_kernel,
        out_shape=[
            jax.ShapeDtypeStruct((b_pad, 3), jnp.float32),
            jax.ShapeDtypeStruct((b_pad, 3), jnp.float32),
            jax.ShapeDtypeStruct((b_pad, _T), jnp.float32),
        ],
        grid=(n_blocks,),
        in_specs=[
            pl.BlockSpec((_B_BLK * _RPC, 128), lambda i: (i, 0)),
            pl.BlockSpec((128, 128), lambda i: (0, 0)),
            pl.BlockSpec((1, 128), lambda i: (0, 0)),
            pl.BlockSpec((128, 128), lambda i: (0, 0)),
        ],
        out_specs=[
            pl.BlockSpec((_B_BLK, 3), lambda i: (i, 0)),
            pl.BlockSpec((_B_BLK, 3), lambda i: (i, 0)),
            pl.BlockSpec((_B_BLK, _T), lambda i: (i, 0)),
        ],
        compiler_params=pltpu.CompilerParams(dimension_semantics=("parallel",)),
        cost_estimate=pl.CostEstimate(flops=flops, transcendentals=0,
                                      bytes_accessed=bytes_accessed),
    )(xp, w1_blk, b1_t, wh_stack)

    return {"tr_pred": tr[:b], "rot_pred": rot[:b], "tor_pred": tor[:b]}


def kernel(x, w1_aug, w_heads):
    return _forward(x, w1_aug, w_heads)


# speedup vs baseline: 1.1016x; 1.1016x over previous
import jax
import jax.numpy as jnp
from jax.experimental import pallas as pl
from jax.experimental.pallas import tpu as pltpu

_N = 8
_D = 32
_H = 32
_T = 4
_B_BLK = 1024
_Q = 4
_BQ = _B_BLK // _Q


def _score_kernel(x0_ref, x1_ref, x2_ref, x3_ref, w1a_ref, wh_ref,
                  tr_ref, rot_ref, tor_ref):
    w1 = w1a_ref[0:_D, :]
    b1 = w1a_ref[_D:_D + 1, :]
    wh = wh_ref[...]
    for q, x_ref in enumerate((x0_ref, x1_ref, x2_ref, x3_ref)):
        xv = x_ref[...].reshape(_BQ * _N, _D)
        h = jnp.maximum(
            jnp.dot(xv, w1, preferred_element_type=jnp.float32) + b1, 0.0)
        pooled = jnp.sum(h.reshape(_BQ, _N, _H), axis=1)
        out = jnp.dot(pooled, wh, preferred_element_type=jnp.float32)
        sl = pl.ds(q * _BQ, _BQ)
        tr_ref[sl, :] = out[:, 0:3]
        rot_ref[sl, :] = out[:, 3:6]
        tor_ref[sl, :] = out[:, 6:6 + _T]


@jax.jit
def _forward(x, w1_aug, w_heads):
    b = x.shape[0]
    n_blocks = pl.cdiv(b, _B_BLK)
    b_pad = n_blocks * _B_BLK
    if b_pad != b:
        x = jnp.pad(x, ((0, b_pad - b), (0, 0), (0, 0)))

    rows = b_pad * _N
    flops = 2 * rows * _D * _H + 2 * b_pad * _H * 128
    bytes_accessed = 4 * (rows * _D + (_D + 1) * _H + _H * 128 + b_pad * (3 + 3 + _T))

    def xspec(q):
        return pl.BlockSpec((_BQ, _N, _D), lambda i, q=q: (i * _Q + q, 0, 0))

    tr, rot, tor = pl.pallas_call(
        _score_kernel,
        out_shape=[
            jax.ShapeDtypeStruct((b_pad, 3), jnp.float32),
            jax.ShapeDtypeStruct((b_pad, 3), jnp.float32),
            jax.ShapeDtypeStruct((b_pad, _T), jnp.float32),
        ],
        grid=(n_blocks,),
        in_specs=[
            xspec(0), xspec(1), xspec(2), xspec(3),
            pl.BlockSpec((_D + 1, _H), lambda i: (0, 0)),
            pl.BlockSpec((_H, 128), lambda i: (0, 0)),
        ],
        out_specs=[
            pl.BlockSpec((_B_BLK, 3), lambda i: (i, 0)),
            pl.BlockSpec((_B_BLK, 3), lambda i: (i, 0)),
            pl.BlockSpec((_B_BLK, _T), lambda i: (i, 0)),
        ],
        compiler_params=pltpu.CompilerParams(dimension_semantics=("parallel",)),
        cost_estimate=pl.CostEstimate(flops=flops, transcendentals=0,
                                      bytes_accessed=bytes_accessed),
    )(x, x, x, x, w1_aug, w_heads)

    if b_pad != b:
        tr, rot, tor = tr[:b], rot[:b], tor[:b]
    return {"tr_pred": tr, "rot_pred": rot, "tor_pred": tor}


def kernel(x, w1_aug, w_heads):
    return _forward(x, w1_aug, w_heads)


# speedup vs baseline: 1.1279x; 1.0239x over previous
import jax
import jax.numpy as jnp
from jax.experimental import pallas as pl
from jax.experimental.pallas import tpu as pltpu

_N = 8
_D = 32
_H = 32
_T = 4
_B_BLK = 1024


def _score_kernel(x_ref, w1a_ref, wh_ref, tr_ref, rot_ref, tor_ref):
    w1 = w1a_ref[0:_D, :]
    b1 = w1a_ref[_D:_D + 1, :]
    xv = x_ref[...].reshape(_B_BLK * _N, _D)
    h = jnp.maximum(
        jnp.dot(xv, w1, preferred_element_type=jnp.float32) + b1, 0.0)
    pooled = jnp.sum(h.reshape(_B_BLK, _N, _H), axis=1)
    out = jnp.dot(pooled, wh_ref[...], preferred_element_type=jnp.float32)
    tr_ref[...] = out[:, 0:3]
    rot_ref[...] = out[:, 3:6]
    tor_ref[...] = out[:, 6:6 + _T]


@jax.jit
def _forward(x, w1_aug, w_heads):
    b = x.shape[0]
    n_blocks = pl.cdiv(b, _B_BLK)
    b_pad = n_blocks * _B_BLK
    if b_pad != b:
        x = jnp.pad(x, ((0, b_pad - b), (0, 0), (0, 0)))

    rows = b_pad * _N
    flops = 2 * rows * _D * _H + 2 * b_pad * _H * 128
    bytes_accessed = 4 * (rows * _D + (_D + 1) * _H + _H * 128 + b_pad * (3 + 3 + _T))

    tr, rot, tor = pl.pallas_call(
        _score_kernel,
        out_shape=[
            jax.ShapeDtypeStruct((b_pad, 3), jnp.float32),
            jax.ShapeDtypeStruct((b_pad, 3), jnp.float32),
            jax.ShapeDtypeStruct((b_pad, _T), jnp.float32),
        ],
        grid=(n_blocks,),
        in_specs=[
            pl.BlockSpec((_B_BLK, _N, _D), lambda i: (i, 0, 0)),
            pl.BlockSpec((_D + 1, _H), lambda i: (0, 0)),
            pl.BlockSpec((_H, 128), lambda i: (0, 0)),
        ],
        out_specs=[
            pl.BlockSpec((_B_BLK, 3), lambda i: (i, 0)),
            pl.BlockSpec((_B_BLK, 3), lambda i: (i, 0)),
            pl.BlockSpec((_B_BLK, _T), lambda i: (i, 0)),
        ],
        compiler_params=pltpu.CompilerParams(dimension_semantics=("parallel",)),
        cost_estimate=pl.CostEstimate(flops=flops, transcendentals=0,
                                      bytes_accessed=bytes_accessed),
    )(x, w1_aug, w_heads)

    if b_pad != b:
        tr, rot, tor = tr[:b], rot[:b], tor[:b]
    return {"tr_pred": tr, "rot_pred": rot, "tor_pred": tor}


def kernel(x, w1_aug, w_heads):
    return _forward(x, w1_aug, w_heads)


# speedup vs baseline: 1.2007x; 1.0645x over previous
import jax
import jax.numpy as jnp
from jax.experimental import pallas as pl
from jax.experimental.pallas import tpu as pltpu

_N = 8
_D = 32
_H = 32
_T = 4
_B_BLK = 2048


def _score_kernel(x_ref, w1a_ref, wh_ref, tr_ref, rot_ref, tor_ref):
    w1 = w1a_ref[0:_D, :]
    b1 = w1a_ref[_D:_D + 1, :]
    xv = x_ref[...].reshape(_B_BLK * _N, _D)
    h = jnp.maximum(
        jnp.dot(xv, w1, preferred_element_type=jnp.float32) + b1, 0.0)
    pooled = jnp.sum(h.reshape(_B_BLK, _N, _H), axis=1)
    out = jnp.dot(pooled, wh_ref[...], preferred_element_type=jnp.float32)
    tr_ref[...] = out[:, 0:3]
    rot_ref[...] = out[:, 3:6]
    tor_ref[...] = out[:, 6:6 + _T]


@jax.jit
def _forward(x, w1_aug, w_heads):
    b = x.shape[0]
    n_blocks = pl.cdiv(b, _B_BLK)
    b_pad = n_blocks * _B_BLK
    if b_pad != b:
        x = jnp.pad(x, ((0, b_pad - b), (0, 0), (0, 0)))

    rows = b_pad * _N
    flops = 2 * rows * _D * _H + 2 * b_pad * _H * 128
    bytes_accessed = 4 * (rows * _D + (_D + 1) * _H + _H * 128 + b_pad * (3 + 3 + _T))

    tr, rot, tor = pl.pallas_call(
        _score_kernel,
        out_shape=[
            jax.ShapeDtypeStruct((b_pad, 3), jnp.float32),
            jax.ShapeDtypeStruct((b_pad, 3), jnp.float32),
            jax.ShapeDtypeStruct((b_pad, _T), jnp.float32),
        ],
        grid=(n_blocks,),
        in_specs=[
            pl.BlockSpec((_B_BLK, _N, _D), lambda i: (i, 0, 0)),
            pl.BlockSpec((_D + 1, _H), lambda i: (0, 0)),
            pl.BlockSpec((_H, 128), lambda i: (0, 0)),
        ],
        out_specs=[
            pl.BlockSpec((_B_BLK, 3), lambda i: (i, 0)),
            pl.BlockSpec((_B_BLK, 3), lambda i: (i, 0)),
            pl.BlockSpec((_B_BLK, _T), lambda i: (i, 0)),
        ],
        compiler_params=pltpu.CompilerParams(dimension_semantics=("parallel",)),
        cost_estimate=pl.CostEstimate(flops=flops, transcendentals=0,
                                      bytes_accessed=bytes_accessed),
    )(x, w1_aug, w_heads)

    if b_pad != b:
        tr, rot, tor = tr[:b], rot[:b], tor[:b]
    return {"tr_pred": tr, "rot_pred": rot, "tor_pred": tor}


def kernel(x, w1_aug, w_heads):
    return _forward(x, w1_aug, w_heads)


# speedup vs baseline: 1.2239x; 1.0193x over previous
import jax
import jax.numpy as jnp
from jax.experimental import pallas as pl
from jax.experimental.pallas import tpu as pltpu

_N = 8
_D = 32
_H = 32
_T = 4
_B_BLK = 4096


def _score_kernel(x_ref, w1a_ref, wh_ref, tr_ref, rot_ref, tor_ref):
    w1 = w1a_ref[0:_D, :]
    b1 = w1a_ref[_D:_D + 1, :]
    xv = x_ref[...].reshape(_B_BLK * _N, _D)
    h = jnp.maximum(
        jnp.dot(xv, w1, preferred_element_type=jnp.float32) + b1, 0.0)
    pooled = jnp.sum(h.reshape(_B_BLK, _N, _H), axis=1)
    out = jnp.dot(pooled, wh_ref[...], preferred_element_type=jnp.float32)
    tr_ref[...] = out[:, 0:3]
    rot_ref[...] = out[:, 3:6]
    tor_ref[...] = out[:, 6:6 + _T]


@jax.jit
def _forward(x, w1_aug, w_heads):
    b = x.shape[0]
    n_blocks = pl.cdiv(b, _B_BLK)
    b_pad = n_blocks * _B_BLK
    if b_pad != b:
        x = jnp.pad(x, ((0, b_pad - b), (0, 0), (0, 0)))

    rows = b_pad * _N
    flops = 2 * rows * _D * _H + 2 * b_pad * _H * 128
    bytes_accessed = 4 * (rows * _D + (_D + 1) * _H + _H * 128 + b_pad * (3 + 3 + _T))

    tr, rot, tor = pl.pallas_call(
        _score_kernel,
        out_shape=[
            jax.ShapeDtypeStruct((b_pad, 3), jnp.float32),
            jax.ShapeDtypeStruct((b_pad, 3), jnp.float32),
            jax.ShapeDtypeStruct((b_pad, _T), jnp.float32),
        ],
        grid=(n_blocks,),
        in_specs=[
            pl.BlockSpec((_B_BLK, _N, _D), lambda i: (i, 0, 0)),
            pl.BlockSpec((_D + 1, _H), lambda i: (0, 0)),
            pl.BlockSpec((_H, 128), lambda i: (0, 0)),
        ],
        out_specs=[
            pl.BlockSpec((_B_BLK, 3), lambda i: (i, 0)),
            pl.BlockSpec((_B_BLK, 3), lambda i: (i, 0)),
            pl.BlockSpec((_B_BLK, _T), lambda i: (i, 0)),
        ],
        compiler_params=pltpu.CompilerParams(dimension_semantics=("parallel",)),
        cost_estimate=pl.CostEstimate(flops=flops, transcendentals=0,
                                      bytes_accessed=bytes_accessed),
    )(x, w1_aug, w_heads)

    if b_pad != b:
        tr, rot, tor = tr[:b], rot[:b], tor[:b]
    return {"tr_pred": tr, "rot_pred": rot, "tor_pred": tor}


def kernel(x, w1_aug, w_heads):
    return _forward(x, w1_aug, w_heads)


# speedup vs baseline: 1.5315x; 1.2513x over previous
import jax
import jax.numpy as jnp
from jax.experimental import pallas as pl
from jax.experimental.pallas import tpu as pltpu

_N = 8
_D = 32
_H = 32
_T = 4
_B_BLK = 4096
_HO = 16


def _score_kernel(x_ref, w1a_ref, wh_ref, out_ref):
    w1 = w1a_ref[0:_D, :]
    b1 = w1a_ref[_D:_D + 1, :]
    xv = x_ref[...].reshape(_B_BLK * _N, _D)
    h = jnp.maximum(
        jnp.dot(xv, w1, preferred_element_type=jnp.float32) + b1, 0.0)
    pooled = jnp.sum(h.reshape(_B_BLK, _N, _H), axis=1)
    out_ref[...] = jax.lax.dot_general(
        wh_ref[...], pooled, (((0,), (1,)), ((), ())),
        preferred_element_type=jnp.float32)


@jax.jit
def _forward(x, w1_aug, w_heads):
    b = x.shape[0]
    n_blocks = pl.cdiv(b, _B_BLK)
    b_pad = n_blocks * _B_BLK
    if b_pad != b:
        x = jnp.pad(x, ((0, b_pad - b), (0, 0), (0, 0)))

    rows = b_pad * _N
    flops = 2 * rows * _D * _H + 2 * b_pad * _H * 128
    bytes_accessed = 4 * (rows * _D + (_D + 1) * _H + _H * 128 + b_pad * (3 + 3 + _T))

    out_t = pl.pallas_call(
        _score_kernel,
        out_shape=jax.ShapeDtypeStruct((_HO, b_pad), jnp.float32),
        grid=(n_blocks,),
        in_specs=[
            pl.BlockSpec((_B_BLK, _N, _D), lambda i: (i, 0, 0)),
            pl.BlockSpec((_D + 1, _H), lambda i: (0, 0)),
            pl.BlockSpec((_H, _HO), lambda i: (0, 0)),
        ],
        out_specs=pl.BlockSpec((_HO, _B_BLK), lambda i: (0, i)),
        compiler_params=pltpu.CompilerParams(dimension_semantics=("parallel",)),
        cost_estimate=pl.CostEstimate(flops=flops, transcendentals=0,
                                      bytes_accessed=bytes_accessed),
    )(x, w1_aug, w_heads[:, :_HO])

    if b_pad != b:
        out_t = out_t[:, :b]
    return {
        "tr_pred": out_t[0:3].T,
        "rot_pred": out_t[3:6].T,
        "tor_pred": out_t[6:6 + _T].T,
    }


def kernel(x, w1_aug, w_heads):
    return _forward(x, w1_aug, w_heads)
